# baseline (speedup 1.0000x reference)
import functools

import jax
import jax.numpy as jnp
import numpy as np
from jax import lax
from jax.experimental import pallas as pl
from jax.experimental.pallas import tpu as pltpu
from jax.experimental.pallas import tpu_sc as plsc

N_ELEC = 10000
N_NUC = 1000
N_EE = 160000
N_SAME = 80000
N_EN = 160000
N_NN = 16000
EMB = 128
DIM = 128
EDGE_EMB = 16
HID = 32
RBF = 16
N_LAYER = 3

_EDGE_BLK = 2048
PAD_E = 163840
PAD_NN = 16384
_NC = 2
_NS = 16
_SQRT2 = 1.4142135623730951


def _silu(x):
    return x * jax.nn.sigmoid(x)


def _stage_a_body(ee_d_ref, en_d_ref,
                  W1s_ref, b1s_ref, W2s_ref, W1d_ref, b1d_ref, W2d_ref,
                  Wee_ref, bee_ref, W1en_ref, b1en_ref, W2en_ref,
                  Wsc_ref, Ww_ref,
                  vee_ref, gdvT_ref, eenv_ref, sc0_ref, sc1_ref, wedge_ref):
    pid = pl.program_id(0)
    sigma2 = (lax.broadcasted_iota(jnp.int32, (1, RBF), 1).astype(jnp.float32)
              * jnp.float32(4.5 / (RBF - 1)) + 0.5)

    ee_d = ee_d_ref[...]
    r_ee = ee_d[:, 3]
    feats = jnp.exp(-r_ee[:, None] * sigma2)
    row = (pid * _EDGE_BLK
           + lax.broadcasted_iota(jnp.int32, (_EDGE_BLK, 1), 0))
    is_same = row < PAD_E // 2
    hs = _silu(jnp.dot(feats, W1s_ref[...], preferred_element_type=jnp.float32)
               + b1s_ref[...][None, :])
    fs = jnp.dot(hs, W2s_ref[...], preferred_element_type=jnp.float32)
    hd = _silu(jnp.dot(feats, W1d_ref[...], preferred_element_type=jnp.float32)
               + b1d_ref[...][None, :])
    fd = jnp.dot(hd, W2d_ref[...], preferred_element_type=jnp.float32)
    filt = jnp.where(is_same, fs, fd)
    g_ee = jnp.log1p(r_ee) / (r_ee + 1e-12)
    data = _silu(jnp.dot(ee_d * g_ee[:, None], Wee_ref[...],
                         preferred_element_type=jnp.float32) + bee_ref[...][None, :])
    vee_ref[...] = filt * data

    en_d = en_d_ref[...]
    r_en = en_d[:, 3]
    g_en = jnp.log1p(r_en) / (r_en + 1e-12)
    gdvT_ref[...] = (en_d * g_en[:, None]).T
    eenv_ref[0, 0, :] = jnp.exp(-r_en)
    feats_en = jnp.exp(-r_en[:, None] * sigma2)
    h_en = _silu(jnp.dot(feats_en, W1en_ref[...],
                         preferred_element_type=jnp.float32) + b1en_ref[...][None, :])
    edge16 = jnp.dot(h_en, W2en_ref[...], preferred_element_type=jnp.float32)
    scale = jnp.dot(edge16, Wsc_ref[...], preferred_element_type=jnp.float32)
    sc0_ref[...] = jnp.concatenate([scale[:, 0:64], scale[:, 128:192]], axis=1)
    sc1_ref[...] = jnp.concatenate([scale[:, 64:128], scale[:, 192:256]],
                                   axis=1)
    wedge_ref[...] = jnp.dot(edge16, Ww_ref[...],
                             preferred_element_type=jnp.float32)


def _stage_a(ee_dists, en_dists, W1s, b1s, W2s, W1d, b1d, W2d, Wee, bee,
             W1en, b1en, W2en, Wsc, Ww):
    n_blk = PAD_E // _EDGE_BLK
    blk = _EDGE_BLK
    full = lambda shape: pl.BlockSpec(shape, lambda i: tuple(0 for _ in shape))
    return pl.pallas_call(
        _stage_a_body,
        grid=(n_blk,),
        in_specs=[
            pl.BlockSpec((blk, 4), lambda i: (i, 0)),
            pl.BlockSpec((blk, 4), lambda i: (i, 0)),
            full((RBF, HID)), full((HID,)), full((HID, EMB // 2)),
            full((RBF, HID)), full((HID,)), full((HID, EMB // 2)),
            full((4, EMB // 2)), full((EMB // 2,)),
            full((RBF, HID)), full((HID,)), full((HID, EDGE_EMB)),
            full((EDGE_EMB, 2 * EMB)), full((EDGE_EMB, DIM)),
        ],
        out_specs=[
            pl.BlockSpec((blk, EMB // 2), lambda i: (i, 0)),
            pl.BlockSpec((4, blk), lambda i: (0, i)),
            pl.BlockSpec((1, 1, blk), lambda i: (i, 0, 0)),
            pl.BlockSpec((blk, EMB), lambda i: (i, 0)),
            pl.BlockSpec((blk, EMB), lambda i: (i, 0)),
            pl.BlockSpec((blk, DIM), lambda i: (i, 0)),
        ],
        out_shape=[
            jax.ShapeDtypeStruct((PAD_E, EMB // 2), jnp.float32),
            jax.ShapeDtypeStruct((4, PAD_E), jnp.float32),
            jax.ShapeDtypeStruct((PAD_E // _EDGE_BLK, 1, _EDGE_BLK), jnp.float32),
            jax.ShapeDtypeStruct((PAD_E, EMB), jnp.float32),
            jax.ShapeDtypeStruct((PAD_E, EMB), jnp.float32),
            jax.ShapeDtypeStruct((PAD_E, DIM), jnp.float32),
        ],
    )(ee_dists, en_dists, W1s, b1s, W2s, W1d, b1d, W2d, Wee, bee,
      W1en, b1en, W2en, Wsc, Ww)


def _stage_a2_body(nn_d_ref, out_ref):
    out_ref[...] = jnp.exp(-nn_d_ref[:, 3])


def _stage_a2(nn_dists):
    return pl.pallas_call(
        _stage_a2_body,
        out_shape=jax.ShapeDtypeStruct((PAD_NN,), jnp.float32),
    )(nn_dists)


def _stage_p_body(ee0_ref, ee1_ref, nc_ref, nn_ref, es0_ref, es1_ref,
                  invn_ref, invnn_ref):
    norm = nc_ref[0] + nc_ref[1] + 1.0
    inv = 1.0 / norm
    invn_ref[...] = inv[:N_ELEC]
    es0_ref[...] = ee0_ref[...] * inv[:, None]
    es1_ref[...] = ee1_ref[...] * inv[:, None]
    nn = nn_ref[0, :N_NUC] + nn_ref[1, :N_NUC] + 1.0
    invnn_ref[...] = 1.0 / nn


def _stage_p(ee_acc, norm_acc, nn_acc):
    return pl.pallas_call(
        _stage_p_body,
        out_shape=[
            jax.ShapeDtypeStruct((10240, 64), jnp.float32),
            jax.ShapeDtypeStruct((10240, 64), jnp.float32),
            jax.ShapeDtypeStruct((N_ELEC,), jnp.float32),
            jax.ShapeDtypeStruct((N_NUC,), jnp.float32),
        ],
    )(ee_acc[0], ee_acc[1], norm_acc, nn_acc)


def _stage_e_body(aggE_ref, aggN_ref, invn_ref, invnn_ref,
                  Ws_ref, bs_ref, Wd_ref, bd_ref, bu_ref, Wo_ref, bo_ref,
                  elec_ref, outpre_ref, ud_ref):
    inv = invn_ref[...]
    elec = aggE_ref[...] * inv[:, None]
    elec_ref[...] = elec
    outpre_ref[...] = (jnp.dot(elec, Wo_ref[...],
                               preferred_element_type=jnp.float32)
                       + bo_ref[...][None, :])
    aggN = aggN_ref[...]
    invnn = invnn_ref[...]
    up = aggN[:N_NUC] * invnn[:, None]
    down = aggN[N_NUC:] * invnn[:, None]
    for l in range(N_LAYER):
        su = jnp.dot(up, Ws_ref[l], preferred_element_type=jnp.float32)
        du = jnp.dot(up, Wd_ref[l], preferred_element_type=jnp.float32)
        sd = jnp.dot(down, Ws_ref[l], preferred_element_type=jnp.float32)
        dd = jnp.dot(down, Wd_ref[l], preferred_element_type=jnp.float32)
        bias = bs_ref[l][None, :] + bd_ref[l][None, :]
        pre_u = (su + dd + bias) / _SQRT2 + bu_ref[l]
        pre_d = (sd + du + bias) / _SQRT2 + bu_ref[l]
        up = (up + _silu(pre_u)) / _SQRT2
        down = (down + _silu(pre_d)) / _SQRT2
    ud_ref[...] = jnp.concatenate([up, down], axis=0)


def _stage_e(aggE, aggN, inv_norm, inv_nneigh, Ws, bs, Wd, bd, bias_u,
             W_o, b_o):
    return pl.pallas_call(
        _stage_e_body,
        out_shape=[
            jax.ShapeDtypeStruct((N_ELEC, EMB), jnp.float32),
            jax.ShapeDtypeStruct((N_ELEC, DIM), jnp.float32),
            jax.ShapeDtypeStruct((2 * N_NUC, DIM), jnp.float32),
        ],
    )(aggE, aggN, inv_norm, inv_nneigh, Ws, bs, Wd, bd,
      bias_u, W_o, b_o)


def _stage_g_body(d_ref, invn_ref, elec_ref, outpre_ref,
                  Wf_ref, bf_ref, out_ref):
    diff = d_ref[...] * invn_ref[...][:, None]
    o = _silu(outpre_ref[...] * diff)
    o = _silu(jnp.dot(o, Wf_ref[...], preferred_element_type=jnp.float32)
              + bf_ref[...][None, :])
    out_ref[...] = (elec_ref[...] + o) / _SQRT2


def _stage_g(aggD, inv_norm, elec_emb, out_pre, W_f, b_f):
    return pl.pallas_call(
        _stage_g_body,
        out_shape=jax.ShapeDtypeStruct((N_ELEC, EMB), jnp.float32),
    )(aggD, inv_norm, elec_emb, out_pre, W_f, b_f)


_CHUNK = 128
_EPT = PAD_E // (_NC * _NS)
_NN_EPT = PAD_NN // (_NC * _NS)


def _stage_b_body(vee, iee, eenv, nidx, eidx, nnenv, nncol, nnrow, chg,
                  outS, outN, outM,
                  accS, accN, accM, rows_v, idx_v, idx2_v, val_v, chv_v,
                  zbuf, zbuf1, sem):
    c = lax.axis_index("c")
    s = lax.axis_index("s")
    w = c * _NS + s

    zero16 = jnp.zeros((16,), jnp.float32)

    def zrow(i, carry):
        for j in range(4):
            zbuf[i, pl.ds(j * 16, 16)] = zero16
        return carry

    lax.fori_loop(0, 640, zrow, 0)

    def zrow1(i, carry):
        zbuf1[pl.ds(i * 16, 16)] = zero16
        return carry

    lax.fori_loop(0, 40, zrow1, 0)

    pltpu.sync_copy(zbuf, accS.at[pl.ds(s * 640, 640)])
    pltpu.sync_copy(zbuf1, accN.at[pl.ds(s * 640, 640)])
    pltpu.sync_copy(zbuf1.at[pl.ds(0, 64)], accM.at[pl.ds(s * 64, 64)])
    plsc.subcore_barrier()

    base = w * _EPT

    def ee_chunk(i, carry):
        b = base + i * _CHUNK
        pltpu.sync_copy(vee.at[pl.ds(b, _CHUNK)], rows_v)
        pltpu.sync_copy(iee.at[pl.ds(b, _CHUNK)], idx_v)
        pltpu.sync_copy(rows_v, accS.at[idx_v], add=True)
        return carry

    lax.fori_loop(0, _EPT // _CHUNK, ee_chunk, 0)

    def en_chunk(i, carry):
        b = base + i * _CHUNK
        pltpu.sync_copy(eenv.at[pl.ds(b, _CHUNK)], val_v)
        pltpu.sync_copy(nidx.at[pl.ds(b, _CHUNK)], idx_v)
        pltpu.sync_copy(eidx.at[pl.ds(b, _CHUNK)], idx2_v)
        pltpu.async_copy(chg.at[idx_v], chv_v, sem).wait()
        for j in range(_CHUNK // 16):
            sl = pl.ds(j * 16, 16)
            val_v[sl] = val_v[sl] * chv_v[sl]
        pltpu.sync_copy(val_v, accN.at[idx2_v], add=True)
        return carry

    lax.fori_loop(0, _EPT // _CHUNK, en_chunk, 0)

    nn_base = w * _NN_EPT

    def nn_chunk(i, carry):
        b = nn_base + i * _CHUNK
        pltpu.sync_copy(nnenv.at[pl.ds(b, _CHUNK)], val_v)
        pltpu.sync_copy(nncol.at[pl.ds(b, _CHUNK)], idx_v)
        pltpu.sync_copy(nnrow.at[pl.ds(b, _CHUNK)], idx2_v)
        pltpu.async_copy(chg.at[idx_v], chv_v, sem).wait()
        for j in range(_CHUNK // 16):
            sl = pl.ds(j * 16, 16)
            val_v[sl] = val_v[sl] * chv_v[sl]
        pltpu.sync_copy(val_v, accM.at[idx2_v], add=True)
        return carry

    lax.fori_loop(0, _NN_EPT // _CHUNK, nn_chunk, 0)

    plsc.subcore_barrier()
    pltpu.sync_copy(accS.at[pl.ds(s * 640, 640)],
                    outS.at[c, pl.ds(s * 640, 640)])
    pltpu.sync_copy(accN.at[pl.ds(s * 640, 640)], zbuf1)
    pltpu.sync_copy(zbuf1, outN.at[c, pl.ds(s * 640, 640)])
    pltpu.sync_copy(accM.at[pl.ds(s * 64, 64)], zbuf1.at[pl.ds(0, 64)])
    pltpu.sync_copy(zbuf1.at[pl.ds(0, 64)], outM.at[c, pl.ds(s * 64, 64)])


def _stage_b_sc(v_ee, idx_ee, e_env, n_idx, e_idx, nn_env, nn_col, nn_row,
                charges_p):
    f32 = jnp.float32
    mesh = plsc.VectorSubcoreMesh(core_axis_name="c", subcore_axis_name="s")
    fn = pl.kernel(
        _stage_b_body,
        mesh=mesh,
        compiler_params=pltpu.CompilerParams(use_tc_tiling_on_sc=False),
        out_type=[
            jax.ShapeDtypeStruct((2, 10240, 64), f32),
            jax.ShapeDtypeStruct((2, 10240), f32),
            jax.ShapeDtypeStruct((2, 1024), f32),
        ],
        scratch_types=[
            pltpu.VMEM_SHARED((10240, 64), f32),
            pltpu.VMEM_SHARED((10240,), f32),
            pltpu.VMEM_SHARED((1024,), f32),
            pltpu.VMEM((_CHUNK, 64), f32),
            pltpu.VMEM((_CHUNK,), jnp.int32),
            pltpu.VMEM((_CHUNK,), jnp.int32),
            pltpu.VMEM((_CHUNK,), f32),
            pltpu.VMEM((_CHUNK,), f32),
            pltpu.VMEM((640, 64), f32),
            pltpu.VMEM((640,), f32),
            pltpu.SemaphoreType.DMA,
        ],
    )
    return fn(v_ee, idx_ee, e_env, n_idx, e_idx, nn_env, nn_col, nn_row,
              charges_p)


def _stage_cd_body(gdvT, kb0, kb1, es0, es1, sc0, sc1, ipack,
                   outE, outN,
                   accE, accN2, kbb, esb, scb, gb, ib3, p0b, p1b, sem):
    c = lax.axis_index("c")
    s = lax.axis_index("s")
    zero16 = jnp.zeros((16,), jnp.float32)

    def zrow(i, carry):
        for j in range(4):
            p0b[i, pl.ds(j * 16, 16)] = zero16
        return carry

    lax.fori_loop(0, 128, zrow, 0)
    for k in range(4):
        pltpu.sync_copy(p0b, accE.at[pl.ds(s * 625 + k * 128, 128)])
    pltpu.sync_copy(p0b.at[pl.ds(0, 113)],
                    accE.at[pl.ds(s * 625 + 512, 113)])
    pltpu.sync_copy(p0b, accN2.at[pl.ds(s * 128, 128)])
    plsc.subcore_barrier()

    def chunk(i, carry):
        b = s * 10240 + i * 128
        pltpu.sync_copy(ipack.at[pl.ds(0, 3), pl.ds(b, 128)], ib3)
        pltpu.sync_copy(gdvT.at[:, pl.ds(b, 128)], gb.at[:, pl.ds(0, 128)])

        @pl.when(c == 0)
        def _():
            h1 = pltpu.async_copy(kb0.at[ib3.at[1]], kbb, sem)
            h2 = pltpu.async_copy(es0.at[ib3.at[0]], esb, sem)
            h3 = pltpu.async_copy(sc0.at[pl.ds(b, 128)], scb, sem)
            h1.wait()
            h2.wait()
            h3.wait()

        @pl.when(c == 1)
        def _():
            h1 = pltpu.async_copy(kb1.at[ib3.at[1]], kbb, sem)
            h2 = pltpu.async_copy(es1.at[ib3.at[0]], esb, sem)
            h3 = pltpu.async_copy(sc1.at[pl.ds(b, 128)], scb, sem)
            h1.wait()
            h2.wait()
            h3.wait()

        @plsc.parallel_loop(0, 128, unroll=4)
        def row(r):
            a0 = gb[0, pl.ds(r, 16)][0]
            a1 = gb[1, pl.ds(r, 16)][0]
            a2 = gb[2, pl.ds(r, 16)][0]
            a3 = gb[3, pl.ds(r, 16)][0]
            for j in range(4):
                sl = pl.ds(j * 16, 16)
                x = (kbb[r, pl.ds(256 + j * 16, 16)]
                     + a0 * kbb[r, pl.ds(j * 16, 16)]
                     + a1 * kbb[r, pl.ds(64 + j * 16, 16)]
                     + a2 * kbb[r, pl.ds(128 + j * 16, 16)]
                     + a3 * kbb[r, pl.ds(192 + j * 16, 16)]
                     + esb[r, sl])
                en = x / (1.0 + jnp.exp(-x))
                p0b[r, sl] = en * scb[r, sl]
                p1b[r, sl] = en * scb[r, pl.ds(64 + j * 16, 16)]
        pltpu.sync_copy(p0b, accE.at[ib3.at[0]], add=True)
        pltpu.sync_copy(p1b, accN2.at[ib3.at[2]], add=True)
        return carry

    lax.fori_loop(0, PAD_E // _NS // 128, chunk, 0)
    plsc.subcore_barrier()
    for k in range(4):
        pltpu.sync_copy(accE.at[pl.ds(s * 625 + k * 128, 128)],
                        outE.at[c, pl.ds(s * 625 + k * 128, 128)])
    pltpu.sync_copy(accE.at[pl.ds(s * 625 + 512, 113)],
                    outE.at[c, pl.ds(s * 625 + 512, 113)])
    pltpu.sync_copy(accN2.at[pl.ds(s * 128, 128)],
                    outN.at[c, pl.ds(s * 128, 128)])


def _stage_cd_sc(gdvT, kb0, kb1, es0, es1, sc0, sc1, ipack):
    f32 = jnp.float32
    mesh = plsc.VectorSubcoreMesh(core_axis_name="c", subcore_axis_name="s")
    fn = pl.kernel(
        _stage_cd_body,
        mesh=mesh,
        compiler_params=pltpu.CompilerParams(use_tc_tiling_on_sc=False),
        out_type=[
            jax.ShapeDtypeStruct((2, 10000, 64), f32),
            jax.ShapeDtypeStruct((2, 2048, 64), f32),
        ],
        scratch_types=[
            pltpu.VMEM_SHARED((10000, 64), f32),
            pltpu.VMEM_SHARED((2048, 64), f32),
            pltpu.VMEM((128, 320), f32),
            pltpu.VMEM((128, 64), f32),
            pltpu.VMEM((128, 128), f32),
            pltpu.VMEM((4, 144), f32),
            pltpu.VMEM((3, 128), jnp.int32),
            pltpu.VMEM((128, 64), f32),
            pltpu.VMEM((128, 64), f32),
            pltpu.SemaphoreType.DMA,
        ],
    )
    return fn(gdvT, kb0, kb1, es0, es1, sc0, sc1, ipack)


def _stage_f_body(ud0, ud1, wed, ipack,
                  outD,
                  accD, udb, wb, ib4, pb, sem):
    c = lax.axis_index("c")
    s = lax.axis_index("s")
    zero16 = jnp.zeros((16,), jnp.float32)

    def zrow(i, carry):
        for j in range(4):
            pb[i, pl.ds(j * 16, 16)] = zero16
        return carry

    lax.fori_loop(0, 128, zrow, 0)
    for k in range(4):
        pltpu.sync_copy(pb, accD.at[pl.ds(s * 625 + k * 128, 128)])
    pltpu.sync_copy(pb.at[pl.ds(0, 113)], accD.at[pl.ds(s * 625 + 512, 113)])
    plsc.subcore_barrier()

    def chunk(i, carry):
        b = s * 10240 + i * 128
        pltpu.sync_copy(ipack.at[pl.ds(0, 4), pl.ds(b, 128)], ib4)
        h3 = pltpu.async_copy(wed.at[pl.ds(b, 128)], wb, sem)

        @pl.when(c == 0)
        def _():
            pltpu.async_copy(ud0.at[ib4.at[3]], udb, sem).wait()

        @pl.when(c == 1)
        def _():
            pltpu.async_copy(ud1.at[ib4.at[3]], udb, sem).wait()

        h3.wait()

        @plsc.parallel_loop(0, 128, unroll=2)
        def row(r):
            for j in range(4):
                sl = pl.ds(j * 16, 16)
                pb[r, sl] = udb[r, sl] * wb[r, pl.ds(c * 64 + j * 16, 16)]

        pltpu.sync_copy(pb, accD.at[ib4.at[0]], add=True)
        return carry

    lax.fori_loop(0, PAD_E // _NS // 128, chunk, 0)
    plsc.subcore_barrier()
    for k in range(4):
        pltpu.sync_copy(accD.at[pl.ds(s * 625 + k * 128, 128)],
                        outD.at[c, pl.ds(s * 625 + k * 128, 128)])
    pltpu.sync_copy(accD.at[pl.ds(s * 625 + 512, 113)],
                    outD.at[c, pl.ds(s * 625 + 512, 113)])


def _stage_f_sc(ud0, ud1, w_edge, ipack):
    f32 = jnp.float32
    mesh = plsc.VectorSubcoreMesh(core_axis_name="c", subcore_axis_name="s")
    fn = pl.kernel(
        _stage_f_body,
        mesh=mesh,
        compiler_params=pltpu.CompilerParams(use_tc_tiling_on_sc=False),
        out_type=jax.ShapeDtypeStruct((2, 10000, 64), f32),
        scratch_types=[
            pltpu.VMEM_SHARED((10000, 64), f32),
            pltpu.VMEM((128, 64), f32),
            pltpu.VMEM((128, 128), f32),
            pltpu.VMEM((4, 128), jnp.int32),
            pltpu.VMEM((128, 64), f32),
            pltpu.SemaphoreType.DMA,
        ],
    )
    return fn(ud0, ud1, w_edge, ipack)


def _seg_sum(vals, idx, num):
    return jax.ops.segment_sum(vals, idx, num_segments=num)


def kernel(elec_elec_dists, elec_nuc_dists, nuc_nuc_dists, flat_charges,
           W1_same, b1_same, W2_same, W1_diff, b1_diff, W2_diff, W_ee, b_ee,
           kernel, bias_nuc, W1_en, b1_en, W2_en, W_scale,
           Ws, bs, Wd, bd, bias_u, W_o, b_o, W_w, W_f, b_f,
           elec_elec_idx, elec_nuc_idx_elec, elec_nuc_idx_nuc, nuc_nuc_idx,
           spin_mask):
    f32 = jnp.float32
    pad_d = jnp.asarray([0.0, 0.0, 0.0, 100.0], f32)

    def pad_dists(d, n):
        return jnp.concatenate(
            [d, jnp.broadcast_to(pad_d, (n - d.shape[0], 4))], axis=0)

    ee_d = jnp.concatenate([
        pad_dists(elec_elec_dists[:N_SAME], PAD_E // 2),
        pad_dists(elec_elec_dists[N_SAME:], PAD_E // 2)], axis=0)
    en_d = pad_dists(elec_nuc_dists, PAD_E)
    nn_d = pad_dists(nuc_nuc_dists, PAD_NN)
    v_ee, gdvT, e_env, sc0, sc1, w_edge = _stage_a(
        ee_d, en_d, W1_same, b1_same, W2_same,
        W1_diff, b1_diff, W2_diff, W_ee, b_ee, W1_en, b1_en, W2_en,
        W_scale, W_w)
    e_env = e_env.reshape(PAD_E)
    nn_env = _stage_a2(nn_d)

    i32 = jnp.int32

    def pad_idx(a, n):
        return jnp.concatenate([a.astype(i32), jnp.zeros(n - a.shape[0], i32)])

    e_idx = pad_idx(elec_nuc_idx_elec, PAD_E)
    n_idx = pad_idx(elec_nuc_idx_nuc, PAD_E)
    ee_i = elec_elec_idx[0].astype(i32)

    def pad_idx_v(a, n, v):
        return jnp.concatenate(
            [a.astype(i32), jnp.full(n - a.shape[0], v, i32)])

    idx_ee = jnp.concatenate([pad_idx_v(ee_i[:N_SAME], PAD_E // 2, 10016),
                              pad_idx_v(ee_i[N_SAME:], PAD_E // 2, 10016)])
    mask = spin_mask[e_idx]
    nidx2 = n_idx + N_NUC * mask.astype(i32)
    ud_idx = n_idx + N_NUC * (1 - mask.astype(i32))
    nn_row = pad_idx(nuc_nuc_idx[0], PAD_NN)
    nn_col = pad_idx(nuc_nuc_idx[1], PAD_NN)
    charges_p = jnp.concatenate([flat_charges,
                                 jnp.zeros(1024 - N_NUC, f32)])

    ee_acc, norm_acc, nn_acc = _stage_b_sc(
        v_ee, idx_ee, e_env, n_idx, e_idx, nn_env, nn_col, nn_row, charges_p)

    es0, es1, inv_norm, inv_nneigh = _stage_p(ee_acc, norm_acc, nn_acc)

    def kb_half(c):
        kh = kernel[:, :, 64 * c:64 * (c + 1)]
        kh = jnp.swapaxes(kh, 1, 1).reshape(N_NUC, 256)
        return jnp.concatenate([kh, bias_nuc[:, 64 * c:64 * (c + 1)]], axis=1)

    e_idx_s = e_idx
    n2_s = nidx2
    ipack = jnp.stack([e_idx_s, n_idx, n2_s, ud_idx])
    outE, outN = _stage_cd_sc(gdvT, kb_half(0), kb_half(1), es0, es1,
                              sc0, sc1, ipack)
    aggE = jnp.concatenate([outE[0, :N_ELEC], outE[1, :N_ELEC]], axis=1)
    aggN = jnp.concatenate([outN[0, :2 * N_NUC], outN[1, :2 * N_NUC]], axis=1)

    elec_emb, out_pre, UD = _stage_e(aggE, aggN, inv_norm, inv_nneigh,
                                     Ws, bs, Wd, bd, bias_u, W_o, b_o)

    outD = _stage_f_sc(UD[:, :64], UD[:, 64:], w_edge, ipack)
    aggD = jnp.concatenate([outD[0, :N_ELEC], outD[1, :N_ELEC]], axis=1)

    return _stage_g(aggD, inv_norm, elec_emb, out_pre, W_f, b_f)


# speedup vs baseline: 1.2534x; 1.2534x over previous
import functools

import jax
import jax.numpy as jnp
import numpy as np
from jax import lax
from jax.experimental import pallas as pl
from jax.experimental.pallas import tpu as pltpu
from jax.experimental.pallas import tpu_sc as plsc

N_ELEC = 10000
N_NUC = 1000
N_EE = 160000
N_SAME = 80000
N_EN = 160000
N_NN = 16000
EMB = 128
DIM = 128
EDGE_EMB = 16
HID = 32
RBF = 16
N_LAYER = 3

_EDGE_BLK = 2048
PAD_E = 163840
PAD_NN = 16384
_NC = 2
_NS = 16
_SQRT2 = 1.4142135623730951


def _silu(x):
    return x * jax.nn.sigmoid(x)


def _stage_a_body(ee_d_ref, en_d_ref,
                  W1s_ref, b1s_ref, W2s_ref, W1d_ref, b1d_ref, W2d_ref,
                  Wee_ref, bee_ref, W1en_ref, b1en_ref, W2en_ref,
                  Wsc_ref, Ww_ref,
                  vee_ref, gdvT_ref, eenv_ref, sc0_ref, sc1_ref, wedge_ref):
    pid = pl.program_id(0)
    sigma2 = (lax.broadcasted_iota(jnp.int32, (1, RBF), 1).astype(jnp.float32)
              * jnp.float32(4.5 / (RBF - 1)) + 0.5)

    ee_d = ee_d_ref[...]
    r_ee = ee_d[:, 3]
    feats = jnp.exp(-r_ee[:, None] * sigma2)
    row = (pid * _EDGE_BLK
           + lax.broadcasted_iota(jnp.int32, (_EDGE_BLK, 1), 0))
    is_same = row < PAD_E // 2
    hs = _silu(jnp.dot(feats, W1s_ref[...], preferred_element_type=jnp.float32)
               + b1s_ref[...][None, :])
    fs = jnp.dot(hs, W2s_ref[...], preferred_element_type=jnp.float32)
    hd = _silu(jnp.dot(feats, W1d_ref[...], preferred_element_type=jnp.float32)
               + b1d_ref[...][None, :])
    fd = jnp.dot(hd, W2d_ref[...], preferred_element_type=jnp.float32)
    filt = jnp.where(is_same, fs, fd)
    g_ee = jnp.log1p(r_ee) / (r_ee + 1e-12)
    data = _silu(jnp.dot(ee_d * g_ee[:, None], Wee_ref[...],
                         preferred_element_type=jnp.float32) + bee_ref[...][None, :])
    vee_ref[...] = filt * data

    en_d = en_d_ref[...]
    r_en = en_d[:, 3]
    g_en = jnp.log1p(r_en) / (r_en + 1e-12)
    gdvT_ref[...] = (en_d * g_en[:, None]).T
    eenv_ref[0, 0, :] = jnp.exp(-r_en)
    feats_en = jnp.exp(-r_en[:, None] * sigma2)
    h_en = _silu(jnp.dot(feats_en, W1en_ref[...],
                         preferred_element_type=jnp.float32) + b1en_ref[...][None, :])
    edge16 = jnp.dot(h_en, W2en_ref[...], preferred_element_type=jnp.float32)
    scale = jnp.dot(edge16, Wsc_ref[...], preferred_element_type=jnp.float32)
    sc0_ref[...] = jnp.concatenate([scale[:, 0:64], scale[:, 128:192]], axis=1)
    sc1_ref[...] = jnp.concatenate([scale[:, 64:128], scale[:, 192:256]],
                                   axis=1)
    wedge_ref[...] = jnp.dot(edge16, Ww_ref[...],
                             preferred_element_type=jnp.float32)


def _stage_a(ee_dists, en_dists, W1s, b1s, W2s, W1d, b1d, W2d, Wee, bee,
             W1en, b1en, W2en, Wsc, Ww):
    n_blk = PAD_E // _EDGE_BLK
    blk = _EDGE_BLK
    full = lambda shape: pl.BlockSpec(shape, lambda i: tuple(0 for _ in shape))
    return pl.pallas_call(
        _stage_a_body,
        grid=(n_blk,),
        in_specs=[
            pl.BlockSpec((blk, 4), lambda i: (i, 0)),
            pl.BlockSpec((blk, 4), lambda i: (i, 0)),
            full((RBF, HID)), full((HID,)), full((HID, EMB // 2)),
            full((RBF, HID)), full((HID,)), full((HID, EMB // 2)),
            full((4, EMB // 2)), full((EMB // 2,)),
            full((RBF, HID)), full((HID,)), full((HID, EDGE_EMB)),
            full((EDGE_EMB, 2 * EMB)), full((EDGE_EMB, DIM)),
        ],
        out_specs=[
            pl.BlockSpec((blk, EMB // 2), lambda i: (i, 0)),
            pl.BlockSpec((4, blk), lambda i: (0, i)),
            pl.BlockSpec((1, 1, blk), lambda i: (i, 0, 0)),
            pl.BlockSpec((blk, EMB), lambda i: (i, 0)),
            pl.BlockSpec((blk, EMB), lambda i: (i, 0)),
            pl.BlockSpec((blk, DIM), lambda i: (i, 0)),
        ],
        out_shape=[
            jax.ShapeDtypeStruct((PAD_E, EMB // 2), jnp.float32),
            jax.ShapeDtypeStruct((4, PAD_E), jnp.float32),
            jax.ShapeDtypeStruct((PAD_E // _EDGE_BLK, 1, _EDGE_BLK), jnp.float32),
            jax.ShapeDtypeStruct((PAD_E, EMB), jnp.float32),
            jax.ShapeDtypeStruct((PAD_E, EMB), jnp.float32),
            jax.ShapeDtypeStruct((PAD_E, DIM), jnp.float32),
        ],
    )(ee_dists, en_dists, W1s, b1s, W2s, W1d, b1d, W2d, Wee, bee,
      W1en, b1en, W2en, Wsc, Ww)


def _stage_a2_body(nn_d_ref, out_ref):
    out_ref[...] = jnp.exp(-nn_d_ref[:, 3])


def _stage_a2(nn_dists):
    return pl.pallas_call(
        _stage_a2_body,
        out_shape=jax.ShapeDtypeStruct((PAD_NN,), jnp.float32),
    )(nn_dists)


def _stage_p_body(ee0_ref, ee1_ref, nc_ref, nn_ref, es0_ref, es1_ref,
                  invn_ref, invnn_ref):
    norm = nc_ref[0] + nc_ref[1] + 1.0
    inv = 1.0 / norm
    invn_ref[...] = inv[:N_ELEC]
    es0_ref[...] = ee0_ref[...] * inv[:, None]
    es1_ref[...] = ee1_ref[...] * inv[:, None]
    nn = nn_ref[0, :N_NUC] + nn_ref[1, :N_NUC] + 1.0
    invnn_ref[...] = 1.0 / nn


def _stage_p(ee_acc, norm_acc, nn_acc):
    return pl.pallas_call(
        _stage_p_body,
        out_shape=[
            jax.ShapeDtypeStruct((10240, 64), jnp.float32),
            jax.ShapeDtypeStruct((10240, 64), jnp.float32),
            jax.ShapeDtypeStruct((N_ELEC,), jnp.float32),
            jax.ShapeDtypeStruct((N_NUC,), jnp.float32),
        ],
    )(ee_acc[0], ee_acc[1], norm_acc, nn_acc)


def _stage_e_body(aggE_ref, aggN_ref, invn_ref, invnn_ref,
                  Ws_ref, bs_ref, Wd_ref, bd_ref, bu_ref, Wo_ref, bo_ref,
                  elec_ref, outpre_ref, ud_ref):
    inv = invn_ref[...]
    elec = aggE_ref[...] * inv[:, None]
    elec_ref[...] = elec
    outpre_ref[...] = (jnp.dot(elec, Wo_ref[...],
                               preferred_element_type=jnp.float32)
                       + bo_ref[...][None, :])
    aggN = aggN_ref[...]
    invnn = invnn_ref[...]
    up = aggN[:N_NUC] * invnn[:, None]
    down = aggN[N_NUC:] * invnn[:, None]
    for l in range(N_LAYER):
        su = jnp.dot(up, Ws_ref[l], preferred_element_type=jnp.float32)
        du = jnp.dot(up, Wd_ref[l], preferred_element_type=jnp.float32)
        sd = jnp.dot(down, Ws_ref[l], preferred_element_type=jnp.float32)
        dd = jnp.dot(down, Wd_ref[l], preferred_element_type=jnp.float32)
        bias = bs_ref[l][None, :] + bd_ref[l][None, :]
        pre_u = (su + dd + bias) / _SQRT2 + bu_ref[l]
        pre_d = (sd + du + bias) / _SQRT2 + bu_ref[l]
        up = (up + _silu(pre_u)) / _SQRT2
        down = (down + _silu(pre_d)) / _SQRT2
    ud_ref[...] = jnp.concatenate([up, down], axis=0)


def _stage_e(aggE, aggN, inv_norm, inv_nneigh, Ws, bs, Wd, bd, bias_u,
             W_o, b_o):
    return pl.pallas_call(
        _stage_e_body,
        out_shape=[
            jax.ShapeDtypeStruct((N_ELEC, EMB), jnp.float32),
            jax.ShapeDtypeStruct((N_ELEC, DIM), jnp.float32),
            jax.ShapeDtypeStruct((2 * N_NUC, DIM), jnp.float32),
        ],
    )(aggE, aggN, inv_norm, inv_nneigh, Ws, bs, Wd, bd,
      bias_u, W_o, b_o)


def _stage_g_body(d_ref, invn_ref, elec_ref, outpre_ref,
                  Wf_ref, bf_ref, out_ref):
    diff = d_ref[...] * invn_ref[...][:, None]
    o = _silu(outpre_ref[...] * diff)
    o = _silu(jnp.dot(o, Wf_ref[...], preferred_element_type=jnp.float32)
              + bf_ref[...][None, :])
    out_ref[...] = (elec_ref[...] + o) / _SQRT2


def _stage_g(aggD, inv_norm, elec_emb, out_pre, W_f, b_f):
    return pl.pallas_call(
        _stage_g_body,
        out_shape=jax.ShapeDtypeStruct((N_ELEC, EMB), jnp.float32),
    )(aggD, inv_norm, elec_emb, out_pre, W_f, b_f)


_CHUNK = 128
_EPT = PAD_E // (_NC * _NS)
_NN_EPT = PAD_NN // (_NC * _NS)


def _stage_b_body(vee, iee, eenv, nidx, eidx, nnenv, nncol, nnrow, chg,
                  outS, outN, outM,
                  accS, accN, accM, rows_v, idx_v, idx2_v, val_v, chv_v,
                  zbuf, zbuf1, sem):
    c = lax.axis_index("c")
    s = lax.axis_index("s")
    w = c * _NS + s

    zero16 = jnp.zeros((16,), jnp.float32)

    def zrow(i, carry):
        for j in range(4):
            zbuf[i, pl.ds(j * 16, 16)] = zero16
        return carry

    lax.fori_loop(0, 640, zrow, 0)

    def zrow1(i, carry):
        zbuf1[pl.ds(i * 16, 16)] = zero16
        return carry

    lax.fori_loop(0, 40, zrow1, 0)

    pltpu.sync_copy(zbuf, accS.at[pl.ds(s * 640, 640)])
    pltpu.sync_copy(zbuf1, accN.at[pl.ds(s * 640, 640)])
    pltpu.sync_copy(zbuf1.at[pl.ds(0, 64)], accM.at[pl.ds(s * 64, 64)])
    plsc.subcore_barrier()

    base = w * _EPT

    def ee_chunk(i, carry):
        b = base + i * _CHUNK
        pltpu.sync_copy(vee.at[pl.ds(b, _CHUNK)], rows_v)
        pltpu.sync_copy(iee.at[pl.ds(b, _CHUNK)], idx_v)
        pltpu.sync_copy(rows_v, accS.at[idx_v], add=True)
        return carry

    lax.fori_loop(0, _EPT // _CHUNK, ee_chunk, 0)

    def en_chunk(i, carry):
        b = base + i * _CHUNK
        pltpu.sync_copy(eenv.at[pl.ds(b, _CHUNK)], val_v)
        pltpu.sync_copy(nidx.at[pl.ds(b, _CHUNK)], idx_v)
        pltpu.sync_copy(eidx.at[pl.ds(b, _CHUNK)], idx2_v)
        pltpu.async_copy(chg.at[idx_v], chv_v, sem).wait()
        for j in range(_CHUNK // 16):
            sl = pl.ds(j * 16, 16)
            val_v[sl] = val_v[sl] * chv_v[sl]
        pltpu.sync_copy(val_v, accN.at[idx2_v], add=True)
        return carry

    lax.fori_loop(0, _EPT // _CHUNK, en_chunk, 0)

    nn_base = w * _NN_EPT

    def nn_chunk(i, carry):
        b = nn_base + i * _CHUNK
        pltpu.sync_copy(nnenv.at[pl.ds(b, _CHUNK)], val_v)
        pltpu.sync_copy(nncol.at[pl.ds(b, _CHUNK)], idx_v)
        pltpu.sync_copy(nnrow.at[pl.ds(b, _CHUNK)], idx2_v)
        pltpu.async_copy(chg.at[idx_v], chv_v, sem).wait()
        for j in range(_CHUNK // 16):
            sl = pl.ds(j * 16, 16)
            val_v[sl] = val_v[sl] * chv_v[sl]
        pltpu.sync_copy(val_v, accM.at[idx2_v], add=True)
        return carry

    lax.fori_loop(0, _NN_EPT // _CHUNK, nn_chunk, 0)

    plsc.subcore_barrier()
    pltpu.sync_copy(accS.at[pl.ds(s * 640, 640)],
                    outS.at[c, pl.ds(s * 640, 640)])
    pltpu.sync_copy(accN.at[pl.ds(s * 640, 640)], zbuf1)
    pltpu.sync_copy(zbuf1, outN.at[c, pl.ds(s * 640, 640)])
    pltpu.sync_copy(accM.at[pl.ds(s * 64, 64)], zbuf1.at[pl.ds(0, 64)])
    pltpu.sync_copy(zbuf1.at[pl.ds(0, 64)], outM.at[c, pl.ds(s * 64, 64)])


def _stage_b_sc(v_ee, idx_ee, e_env, n_idx, e_idx, nn_env, nn_col, nn_row,
                charges_p):
    f32 = jnp.float32
    mesh = plsc.VectorSubcoreMesh(core_axis_name="c", subcore_axis_name="s")
    fn = pl.kernel(
        _stage_b_body,
        mesh=mesh,
        compiler_params=pltpu.CompilerParams(use_tc_tiling_on_sc=False),
        out_type=[
            jax.ShapeDtypeStruct((2, 10240, 64), f32),
            jax.ShapeDtypeStruct((2, 10240), f32),
            jax.ShapeDtypeStruct((2, 1024), f32),
        ],
        scratch_types=[
            pltpu.VMEM_SHARED((10240, 64), f32),
            pltpu.VMEM_SHARED((10240,), f32),
            pltpu.VMEM_SHARED((1024,), f32),
            pltpu.VMEM((_CHUNK, 64), f32),
            pltpu.VMEM((_CHUNK,), jnp.int32),
            pltpu.VMEM((_CHUNK,), jnp.int32),
            pltpu.VMEM((_CHUNK,), f32),
            pltpu.VMEM((_CHUNK,), f32),
            pltpu.VMEM((640, 64), f32),
            pltpu.VMEM((640,), f32),
            pltpu.SemaphoreType.DMA,
        ],
    )
    return fn(v_ee, idx_ee, e_env, n_idx, e_idx, nn_env, nn_col, nn_row,
              charges_p)


def _stage_cd_body(gdvT, kb0, kb1, es0, es1, sc0, sc1, ipack,
                   outE, outN,
                   accE, accN2, kbb, esb, scb, gb, ib3, p0b, p1b, sem):
    c = lax.axis_index("c")
    s = lax.axis_index("s")
    zero16 = jnp.zeros((16,), jnp.float32)

    def zrow(i, carry):
        for j in range(4):
            p0b[i, pl.ds(j * 16, 16)] = zero16
        return carry

    lax.fori_loop(0, 128, zrow, 0)
    for k in range(4):
        pltpu.sync_copy(p0b, accE.at[pl.ds(s * 625 + k * 128, 128)])
    pltpu.sync_copy(p0b.at[pl.ds(0, 113)],
                    accE.at[pl.ds(s * 625 + 512, 113)])
    pltpu.sync_copy(p0b, accN2.at[pl.ds(s * 128, 128)])
    plsc.subcore_barrier()

    def chunk(i, carry):
        b = s * 10240 + i * 128
        pltpu.sync_copy(ipack.at[pl.ds(0, 3), pl.ds(b, 128)], ib3)
        pltpu.sync_copy(gdvT.at[:, pl.ds(b, 128)], gb.at[:, pl.ds(0, 128)])

        @pl.when(c == 0)
        def _():
            h1 = pltpu.async_copy(kb0.at[ib3.at[1]], kbb, sem)
            h2 = pltpu.async_copy(es0.at[ib3.at[0]], esb, sem)
            h3 = pltpu.async_copy(sc0.at[pl.ds(b, 128)], scb, sem)
            h1.wait()
            h2.wait()
            h3.wait()

        @pl.when(c == 1)
        def _():
            h1 = pltpu.async_copy(kb1.at[ib3.at[1]], kbb, sem)
            h2 = pltpu.async_copy(es1.at[ib3.at[0]], esb, sem)
            h3 = pltpu.async_copy(sc1.at[pl.ds(b, 128)], scb, sem)
            h1.wait()
            h2.wait()
            h3.wait()

        @plsc.parallel_loop(0, 128, unroll=2)
        def row(r):
            a0 = gb[0, pl.ds(r, 16)][0]
            a1 = gb[1, pl.ds(r, 16)][0]
            a2 = gb[2, pl.ds(r, 16)][0]
            a3 = gb[3, pl.ds(r, 16)][0]
            for j in range(4):
                sl = pl.ds(j * 16, 16)
                x = (kbb[r, pl.ds(256 + j * 16, 16)]
                     + a0 * kbb[r, pl.ds(j * 16, 16)]
                     + a1 * kbb[r, pl.ds(64 + j * 16, 16)]
                     + a2 * kbb[r, pl.ds(128 + j * 16, 16)]
                     + a3 * kbb[r, pl.ds(192 + j * 16, 16)]
                     + esb[r, sl])
                en = x / (1.0 + jnp.exp(-x))
                p0b[r, sl] = en * scb[r, sl]
                p1b[r, sl] = en * scb[r, pl.ds(64 + j * 16, 16)]
        pltpu.sync_copy(p0b, accE.at[ib3.at[0]], add=True)
        pltpu.sync_copy(p1b, accN2.at[ib3.at[2]], add=True)
        return carry

    lax.fori_loop(0, PAD_E // _NS // 128, chunk, 0)
    plsc.subcore_barrier()
    for k in range(4):
        pltpu.sync_copy(accE.at[pl.ds(s * 625 + k * 128, 128)],
                        outE.at[c, pl.ds(s * 625 + k * 128, 128)])
    pltpu.sync_copy(accE.at[pl.ds(s * 625 + 512, 113)],
                    outE.at[c, pl.ds(s * 625 + 512, 113)])
    pltpu.sync_copy(accN2.at[pl.ds(s * 128, 128)],
                    outN.at[c, pl.ds(s * 128, 128)])


def _stage_cd_sc(gdvT, kb0, kb1, es0, es1, sc0, sc1, ipack):
    f32 = jnp.float32
    mesh = plsc.VectorSubcoreMesh(core_axis_name="c", subcore_axis_name="s")
    fn = pl.kernel(
        _stage_cd_body,
        mesh=mesh,
        compiler_params=pltpu.CompilerParams(use_tc_tiling_on_sc=False),
        out_type=[
            jax.ShapeDtypeStruct((2, 10000, 64), f32),
            jax.ShapeDtypeStruct((2, 2048, 64), f32),
        ],
        scratch_types=[
            pltpu.VMEM_SHARED((10000, 64), f32),
            pltpu.VMEM_SHARED((2048, 64), f32),
            pltpu.VMEM((128, 320), f32),
            pltpu.VMEM((128, 64), f32),
            pltpu.VMEM((128, 128), f32),
            pltpu.VMEM((4, 144), f32),
            pltpu.VMEM((3, 128), jnp.int32),
            pltpu.VMEM((128, 64), f32),
            pltpu.VMEM((128, 64), f32),
            pltpu.SemaphoreType.DMA,
        ],
    )
    return fn(gdvT, kb0, kb1, es0, es1, sc0, sc1, ipack)


def _stage_f_body(ud0, ud1, wed, ipack,
                  outD,
                  accD, udb, wb, ib4, pb, sem):
    c = lax.axis_index("c")
    s = lax.axis_index("s")
    zero16 = jnp.zeros((16,), jnp.float32)

    def zrow(i, carry):
        for j in range(4):
            pb[i, pl.ds(j * 16, 16)] = zero16
        return carry

    lax.fori_loop(0, 128, zrow, 0)
    for k in range(4):
        pltpu.sync_copy(pb, accD.at[pl.ds(s * 625 + k * 128, 128)])
    pltpu.sync_copy(pb.at[pl.ds(0, 113)], accD.at[pl.ds(s * 625 + 512, 113)])
    plsc.subcore_barrier()

    def chunk(i, carry):
        b = s * 10240 + i * 128
        pltpu.sync_copy(ipack.at[pl.ds(0, 4), pl.ds(b, 128)], ib4)
        h3 = pltpu.async_copy(wed.at[pl.ds(b, 128)], wb, sem)

        @pl.when(c == 0)
        def _():
            pltpu.async_copy(ud0.at[ib4.at[3]], udb, sem).wait()

        @pl.when(c == 1)
        def _():
            pltpu.async_copy(ud1.at[ib4.at[3]], udb, sem).wait()

        h3.wait()

        @plsc.parallel_loop(0, 128, unroll=2)
        def row(r):
            for j in range(4):
                sl = pl.ds(j * 16, 16)
                pb[r, sl] = udb[r, sl] * wb[r, pl.ds(c * 64 + j * 16, 16)]

        pltpu.sync_copy(pb, accD.at[ib4.at[0]], add=True)
        return carry

    lax.fori_loop(0, PAD_E // _NS // 128, chunk, 0)
    plsc.subcore_barrier()
    for k in range(4):
        pltpu.sync_copy(accD.at[pl.ds(s * 625 + k * 128, 128)],
                        outD.at[c, pl.ds(s * 625 + k * 128, 128)])
    pltpu.sync_copy(accD.at[pl.ds(s * 625 + 512, 113)],
                    outD.at[c, pl.ds(s * 625 + 512, 113)])


def _stage_f_sc(ud0, ud1, w_edge, ipack):
    f32 = jnp.float32
    mesh = plsc.VectorSubcoreMesh(core_axis_name="c", subcore_axis_name="s")
    fn = pl.kernel(
        _stage_f_body,
        mesh=mesh,
        compiler_params=pltpu.CompilerParams(use_tc_tiling_on_sc=False),
        out_type=jax.ShapeDtypeStruct((2, 10000, 64), f32),
        scratch_types=[
            pltpu.VMEM_SHARED((10000, 64), f32),
            pltpu.VMEM((128, 64), f32),
            pltpu.VMEM((128, 128), f32),
            pltpu.VMEM((4, 128), jnp.int32),
            pltpu.VMEM((128, 64), f32),
            pltpu.SemaphoreType.DMA,
        ],
    )
    return fn(ud0, ud1, w_edge, ipack)


def _seg_sum(vals, idx, num):
    return jax.ops.segment_sum(vals, idx, num_segments=num)


def kernel(elec_elec_dists, elec_nuc_dists, nuc_nuc_dists, flat_charges,
           W1_same, b1_same, W2_same, W1_diff, b1_diff, W2_diff, W_ee, b_ee,
           kernel, bias_nuc, W1_en, b1_en, W2_en, W_scale,
           Ws, bs, Wd, bd, bias_u, W_o, b_o, W_w, W_f, b_f,
           elec_elec_idx, elec_nuc_idx_elec, elec_nuc_idx_nuc, nuc_nuc_idx,
           spin_mask):
    f32 = jnp.float32
    pad_d = jnp.asarray([0.0, 0.0, 0.0, 100.0], f32)

    def pad_dists(d, n):
        return jnp.concatenate(
            [d, jnp.broadcast_to(pad_d, (n - d.shape[0], 4))], axis=0)

    ee_d = jnp.concatenate([
        pad_dists(elec_elec_dists[:N_SAME], PAD_E // 2),
        pad_dists(elec_elec_dists[N_SAME:], PAD_E // 2)], axis=0)
    en_d = pad_dists(elec_nuc_dists, PAD_E)
    nn_d = pad_dists(nuc_nuc_dists, PAD_NN)
    v_ee, gdvT, e_env, sc0, sc1, w_edge = _stage_a(
        ee_d, en_d, W1_same, b1_same, W2_same,
        W1_diff, b1_diff, W2_diff, W_ee, b_ee, W1_en, b1_en, W2_en,
        W_scale, W_w)
    e_env = e_env.reshape(PAD_E)
    nn_env = _stage_a2(nn_d)

    i32 = jnp.int32

    def pad_idx(a, n):
        return jnp.concatenate([a.astype(i32), jnp.zeros(n - a.shape[0], i32)])

    e_idx = pad_idx(elec_nuc_idx_elec, PAD_E)
    n_idx = pad_idx(elec_nuc_idx_nuc, PAD_E)
    ee_i = elec_elec_idx[0].astype(i32)

    def pad_idx_v(a, n, v):
        return jnp.concatenate(
            [a.astype(i32), jnp.full(n - a.shape[0], v, i32)])

    idx_ee = jnp.concatenate([pad_idx_v(ee_i[:N_SAME], PAD_E // 2, 10016),
                              pad_idx_v(ee_i[N_SAME:], PAD_E // 2, 10016)])
    mask = spin_mask[e_idx]
    nidx2 = n_idx + N_NUC * mask.astype(i32)
    ud_idx = n_idx + N_NUC * (1 - mask.astype(i32))
    nn_row = pad_idx(nuc_nuc_idx[0], PAD_NN)
    nn_col = pad_idx(nuc_nuc_idx[1], PAD_NN)
    charges_p = jnp.concatenate([flat_charges,
                                 jnp.zeros(1024 - N_NUC, f32)])

    ee_acc, norm_acc, nn_acc = _stage_b_sc(
        v_ee, idx_ee, e_env, n_idx, e_idx, nn_env, nn_col, nn_row, charges_p)

    es0, es1, inv_norm, inv_nneigh = _stage_p(ee_acc, norm_acc, nn_acc)

    def kb_half(c):
        kh = kernel[:, :, 64 * c:64 * (c + 1)]
        kh = jnp.swapaxes(kh, 1, 1).reshape(N_NUC, 256)
        return jnp.concatenate([kh, bias_nuc[:, 64 * c:64 * (c + 1)]], axis=1)

    e_idx_s = e_idx
    n2_s = nidx2
    ipack = jnp.stack([e_idx_s, n_idx, n2_s, ud_idx])
    outE, outN = _stage_cd_sc(gdvT, kb_half(0), kb_half(1), es0, es1,
                              sc0, sc1, ipack)
    aggE = jnp.concatenate([outE[0, :N_ELEC], outE[1, :N_ELEC]], axis=1)
    aggN = jnp.concatenate([outN[0, :2 * N_NUC], outN[1, :2 * N_NUC]], axis=1)

    elec_emb, out_pre, UD = _stage_e(aggE, aggN, inv_norm, inv_nneigh,
                                     Ws, bs, Wd, bd, bias_u, W_o, b_o)

    outD = _stage_f_sc(UD[:, :64], UD[:, 64:], w_edge, ipack)
    aggD = jnp.concatenate([outD[0, :N_ELEC], outD[1, :N_ELEC]], axis=1)

    return _stage_g(aggD, inv_norm, elec_emb, out_pre, W_f, b_f)
